# baseline (speedup 1.0000x reference)
import functools
from math import sqrt

import jax
import jax.numpy as jnp
from jax.experimental import pallas as pl


K_TOP = 24
NEG_INF = -1e30


def _cross_ranker_kernel(q_ref, k_ref, out_ref, avg_ref):
    q = q_ref[0]
    k = k_ref[0]
    scale = 1.0 / sqrt(q.shape[-1])

    scores = jax.lax.dot_general(
        q, k, (((1,), (1,)), ((), ())),
        preferred_element_type=jnp.float32)
    scores = scores * scale
    scores = scores - jnp.max(scores, axis=-1, keepdims=True)
    e = jnp.exp(scores)
    probs = e / jnp.sum(e, axis=-1, keepdims=True)

    avg = jnp.mean(probs, axis=0, keepdims=True)
    avg_ref[0] = avg

    iota = jax.lax.broadcasted_iota(jnp.int32, avg.shape, 1)
    vals = avg
    top_vals = []
    top_idxs = []
    for _ in range(K_TOP):
        m = jnp.max(vals)
        i = jnp.argmax(vals[0]).astype(jnp.int32)
        top_vals.append(m)
        top_idxs.append(i)
        vals = jnp.where(iota == i, NEG_INF, vals)

    mx = top_vals[0]
    exps = [jnp.exp(v - mx) for v in top_vals]
    denom = functools.reduce(lambda a, b: a + b, exps)
    inv = 1.0 / denom

    for j in range(K_TOP):
        w = exps[j] * inv
        row = k_ref[0, pl.ds(top_idxs[j], 1), :]
        out_ref[0, pl.ds(j, 1), :] = row * w


def kernel(queries, keys):
    B, L, D = queries.shape
    S = keys.shape[1]
    out, avg = pl.pallas_call(
        _cross_ranker_kernel,
        grid=(B,),
        in_specs=[
            pl.BlockSpec((1, L, D), lambda b: (b, 0, 0)),
            pl.BlockSpec((1, S, D), lambda b: (b, 0, 0)),
        ],
        out_specs=[
            pl.BlockSpec((1, K_TOP, D), lambda b: (b, 0, 0)),
            pl.BlockSpec((1, 1, S), lambda b: (b, 0, 0)),
        ],
        out_shape=[
            jax.ShapeDtypeStruct((B, K_TOP, D), jnp.float32),
            jax.ShapeDtypeStruct((B, 1, S), jnp.float32),
        ],
    )(queries, keys)
    return (out, avg.reshape(B, S))
